# Initial kernel scaffold; baseline (speedup 1.0000x reference)
import jax
import jax.numpy as jnp
from jax.experimental import pallas as pl


def kernel(feats, edge_index, W1, attn_l1, attn_r1, W2, attn_l2, attn_r2):
    raise NotImplementedError("write your pallas kernel here")


# speedup vs baseline: 22.5680x; 22.5680x over previous
import dataclasses
import functools

import jax
import jax.numpy as jnp
import numpy as np
from jax import lax
from jax.experimental import pallas as pl
from jax.experimental.pallas import tpu as pltpu
from jax.experimental.pallas import tpu_sc as plsc

N = 10000
E = 160000
D = 256
H = 8
DH = 32
SLOPE = 0.2

NP = 10240
NC = 2
NS = 16
NW = NC * NS
C = 128
NCHUNK = E // C
KMAX = -(-NCHUNK // NW)
EPW = KMAX * C
ROWS_PER_TILE = NP // NS

_F32 = jnp.float32


def _tc1_body(x_ref, w1_ref, alr_ref, f1a_ref, f1b_ref, elr_ref):
    x = x_ref[...]
    f1 = jnp.dot(x, w1_ref[...], preferred_element_type=_F32)
    t = jnp.dot(f1, alr_ref[...], preferred_element_type=_F32)
    elr_ref[...] = t.T
    f1a_ref[...] = f1[:, :128]
    f1b_ref[...] = f1[:, 128:]


def _tc2_body(s1p_ref, ra_ref, rb_ref, w2_ref, alr_ref, brep_ref,
              h1_ref, f2a_ref, f2b_ref, elr_ref):
    s1 = s1p_ref[0] + s1p_ref[1]
    inv = 1.0 / jnp.where(s1 > 0, s1, 1.0)
    invrep = jnp.dot(inv, brep_ref[...], preferred_element_type=_F32)
    sa = ra_ref[0] + ra_ref[1]
    sb = rb_ref[0] + rb_ref[1]
    h1 = jnp.concatenate([sa * invrep[:, :128], sb * invrep[:, 128:]], axis=1)
    h1 = jnp.maximum(h1, 0.0)
    h1_ref[...] = h1
    f2 = jnp.dot(h1, w2_ref[...], preferred_element_type=_F32)
    t = jnp.dot(f2, alr_ref[...], preferred_element_type=_F32)
    elr_ref[...] = t.T
    f2a_ref[...] = f2[:, :128]
    f2b_ref[...] = f2[:, 128:]


def _tc3_body(ra_ref, rb_ref, s2p_ref, brep_ref, out_ref):
    s2 = s2p_ref[0] + s2p_ref[1]
    inv = 1.0 / jnp.where(s2 > 0, s2, 1.0)
    invrep = jnp.dot(inv, brep_ref[...], preferred_element_type=_F32)
    sa = ra_ref[0] + ra_ref[1]
    sb = rb_ref[0] + rb_ref[1]
    out_ref[...] = jnp.concatenate(
        [sa * invrep[:, :128], sb * invrep[:, 128:]], axis=1)


_BLK = 512
_GRID = NP // _BLK


def _tc1(feats_p, w1, alr):
    return pl.pallas_call(
        _tc1_body,
        grid=(_GRID,),
        in_specs=[
            pl.BlockSpec((_BLK, D), lambda i: (i, 0)),
            pl.BlockSpec((D, D), lambda i: (0, 0)),
            pl.BlockSpec((D, 16), lambda i: (0, 0)),
        ],
        out_specs=[
            pl.BlockSpec((_BLK, 128), lambda i: (i, 0)),
            pl.BlockSpec((_BLK, 128), lambda i: (i, 0)),
            pl.BlockSpec((16, _BLK), lambda i: (0, i)),
        ],
        out_shape=[
            jax.ShapeDtypeStruct((NP, 128), _F32),
            jax.ShapeDtypeStruct((NP, 128), _F32),
            jax.ShapeDtypeStruct((16, NP), _F32),
        ],
    )(feats_p, w1, alr)


def _tc2(s1p, r1a, r1b, w2, alr2, brep1):
    return pl.pallas_call(
        _tc2_body,
        grid=(_GRID,),
        in_specs=[
            pl.BlockSpec((2, _BLK, 16), lambda i: (0, i, 0)),
            pl.BlockSpec((2, _BLK, 128), lambda i: (0, i, 0)),
            pl.BlockSpec((2, _BLK, 128), lambda i: (0, i, 0)),
            pl.BlockSpec((D, D), lambda i: (0, 0)),
            pl.BlockSpec((D, 16), lambda i: (0, 0)),
            pl.BlockSpec((16, D), lambda i: (0, 0)),
        ],
        out_specs=[
            pl.BlockSpec((_BLK, D), lambda i: (i, 0)),
            pl.BlockSpec((_BLK, 128), lambda i: (i, 0)),
            pl.BlockSpec((_BLK, 128), lambda i: (i, 0)),
            pl.BlockSpec((16, _BLK), lambda i: (0, i)),
        ],
        out_shape=[
            jax.ShapeDtypeStruct((NP, D), _F32),
            jax.ShapeDtypeStruct((NP, 128), _F32),
            jax.ShapeDtypeStruct((NP, 128), _F32),
            jax.ShapeDtypeStruct((16, NP), _F32),
        ],
    )(s1p, r1a, r1b, w2, alr2, brep1)


def _tc3(r2a, r2b, s2p, brep2):
    return pl.pallas_call(
        _tc3_body,
        grid=(_GRID,),
        in_specs=[
            pl.BlockSpec((2, _BLK, 128), lambda i: (0, i, 0)),
            pl.BlockSpec((2, _BLK, 128), lambda i: (0, i, 0)),
            pl.BlockSpec((2, _BLK, 16), lambda i: (0, i, 0)),
            pl.BlockSpec((16, D), lambda i: (0, 0)),
        ],
        out_specs=pl.BlockSpec((_BLK, D), lambda i: (i, 0)),
        out_shape=jax.ShapeDtypeStruct((NP, D), _F32),
    )(r2a, r2b, s2p, brep2)


_MESH = dict(core_axis_name="c", subcore_axis_name="s")


def _sc_logits(elr_t, src, dst, nheads):
    cp = pltpu.CompilerParams(needs_layout_passes=False,
                              use_tc_tiling_on_sc=False)

    @functools.partial(
        pl.kernel,
        compiler_params=cp,
        out_type=(
            jax.ShapeDtypeStruct((NC, NP, 16), _F32),
            jax.ShapeDtypeStruct((NW, EPW // 8, 128), _F32),
        ),
        mesh=plsc.VectorSubcoreMesh(**_MESH),
        scratch_types=[
            pltpu.VMEM((KMAX, C), jnp.int32),
            pltpu.VMEM((KMAX, C), jnp.int32),
            pltpu.VMEM((EPW // 8, 128), _F32),
            pltpu.VMEM((C, 16), _F32),
            pltpu.VMEM((NP,), _F32),
            pltpu.VMEM_SHARED((NP, 16), _F32),
        ],
    )
    def k(elr_h, src_h, dst_h, s_out, p_out,
          isrc, idst, ploc, sbuf, elr_buf, s_acc):
        cid_core = lax.axis_index("c")
        sid = lax.axis_index("s")
        w = sid * NC + cid_core
        z16 = jnp.zeros((16,), _F32)

        @pl.loop(0, KMAX)
        def _(kk):
            ck = w + NW * kk

            @pl.when(ck < NCHUNK)
            def _():
                pltpu.sync_copy(src_h.at[pl.ds(ck * C, C)], isrc.at[kk])
                pltpu.sync_copy(dst_h.at[pl.ds(ck * C, C)], idst.at[kk])

        @pl.loop(0, EPW // 8)
        def _(r):
            for j in range(8):
                ploc[r, pl.ds(j * 16, 16)] = z16

        @pl.loop(0, C)
        def _(r):
            sbuf[r, :] = z16

        @pl.loop(0, ROWS_PER_TILE // C)
        def _(q):
            off = sid * ROWS_PER_TILE + q * C
            pltpu.sync_copy(sbuf, s_acc.at[pl.ds(off, C)])

        plsc.subcore_barrier()

        for h in range(nheads):
            pltpu.sync_copy(elr_h.at[pl.ds(h * NP, NP)], elr_buf)

            @pl.loop(0, KMAX)
            def _(kk):
                ck = w + NW * kk

                @pl.when(ck < NCHUNK)
                def _():
                    for j in range(C // 16):
                        iv_s = isrc[kk, pl.ds(j * 16, 16)]
                        elv = plsc.load_gather(elr_buf, [iv_s])
                        n = (kk * C + j * 16
                             + lax.iota(jnp.int32, 16))
                        rows = n >> 3
                        cols = (n & 7) * 16 + h
                        plsc.store_scatter(ploc, [rows, cols], elv)

            pltpu.sync_copy(elr_h.at[pl.ds((8 + h) * NP, NP)], elr_buf)

            @pl.loop(0, KMAX)
            def _(kk):
                ck = w + NW * kk

                @pl.when(ck < NCHUNK)
                def _():
                    for j in range(C // 16):
                        iv_d = idst[kk, pl.ds(j * 16, 16)]
                        erv = plsc.load_gather(elr_buf, [iv_d])
                        n = (kk * C + j * 16
                             + lax.iota(jnp.int32, 16))
                        rows = n >> 3
                        cols = (n & 7) * 16 + h
                        elv = plsc.load_gather(ploc, [rows, cols])
                        e = elv + erv
                        e = jnp.maximum(e, e * SLOPE)
                        p = jnp.exp(e)
                        plsc.store_scatter(ploc, [rows, cols], p)

        pltpu.sync_copy(ploc, p_out.at[w])

        @pl.loop(0, KMAX)
        def _(kk):
            ck = w + NW * kk

            @pl.when(ck < NCHUNK)
            def _():
                @pl.loop(0, 16)
                def _(q):
                    for m in range(8):
                        sbuf[q * 8 + m, :] = (
                            ploc[kk * 16 + q, pl.ds(m * 16, 16)])

                pltpu.sync_copy(sbuf, s_acc.at[idst.at[kk]], add=True)

        plsc.subcore_barrier()
        pltpu.sync_copy(s_acc.at[pl.ds(sid * ROWS_PER_TILE, ROWS_PER_TILE)],
                        s_out.at[cid_core,
                                 pl.ds(sid * ROWS_PER_TILE, ROWS_PER_TILE)])

    return k(elr_t.reshape(16 * NP), src, dst)


def _sc_messages(fa, fb, p_hbm, src, dst, hsel_a, hsel_b):
    cp = pltpu.CompilerParams(needs_layout_passes=False,
                              use_tc_tiling_on_sc=False)

    @functools.partial(
        pl.kernel,
        compiler_params=cp,
        out_type=(
            jax.ShapeDtypeStruct((NC, NP, 128), _F32),
            jax.ShapeDtypeStruct((NC, NP, 128), _F32),
        ),
        mesh=plsc.VectorSubcoreMesh(**_MESH),
        scratch_types=[
            pltpu.VMEM((KMAX, C), jnp.int32),
            pltpu.VMEM((KMAX, C), jnp.int32),
            pltpu.VMEM((C, 128), _F32),
            pltpu.VMEM((16, 128), _F32),
            pltpu.VMEM_SHARED((NP, 128), _F32),
        ],
    )
    def k(fa_h, fb_h, p_h, src_h, dst_h, ra_out, rb_out,
          isrc, idst, fbuf, pbuf, m_acc):
        cid_core = lax.axis_index("c")
        sid = lax.axis_index("s")
        w = sid * NC + cid_core
        z16 = jnp.zeros((16,), _F32)

        @pl.loop(0, KMAX)
        def _(kk):
            ck = w + NW * kk

            @pl.when(ck < NCHUNK)
            def _():
                pltpu.sync_copy(src_h.at[pl.ds(ck * C, C)], isrc.at[kk])
                pltpu.sync_copy(dst_h.at[pl.ds(ck * C, C)], idst.at[kk])

        def zero_fbuf():
            @pl.loop(0, C)
            def _(r):
                for j in range(8):
                    fbuf[r, pl.ds(j * 16, 16)] = z16

        def zero_acc():
            @pl.loop(0, ROWS_PER_TILE // C)
            def _(q):
                off = sid * ROWS_PER_TILE + q * C
                pltpu.sync_copy(fbuf, m_acc.at[pl.ds(off, C)])

        zero_fbuf()
        zero_acc()
        plsc.subcore_barrier()

        def message_pass(f_h, hsel, out_ref):
            @pl.loop(0, KMAX)
            def _(kk):
                ck = w + NW * kk

                @pl.when(ck < NCHUNK)
                def _():
                    pltpu.sync_copy(f_h.at[isrc.at[kk]], fbuf)
                    pltpu.sync_copy(p_h.at[w, pl.ds(kk * 16, 16)], pbuf)

                    @pl.loop(0, 16)
                    def _(q):
                        for m in range(8):
                            prow = pbuf[q, pl.ds(m * 16, 16)]
                            r = q * 8 + m
                            for j in range(8):
                                pj = prow[hsel[j]]
                                fbuf[r, pl.ds(j * 16, 16)] = (
                                    fbuf[r, pl.ds(j * 16, 16)] * pj)

                    pltpu.sync_copy(fbuf, m_acc.at[idst.at[kk]], add=True)

            plsc.subcore_barrier()
            pltpu.sync_copy(
                m_acc.at[pl.ds(sid * ROWS_PER_TILE, ROWS_PER_TILE)],
                out_ref.at[cid_core,
                           pl.ds(sid * ROWS_PER_TILE, ROWS_PER_TILE)])

        message_pass(fa_h, hsel_a, ra_out)

        zero_fbuf()
        zero_acc()
        plsc.subcore_barrier()
        message_pass(fb_h, hsel_b, rb_out)

    return k(fa, fb, p_hbm, src, dst)


def _sc_layer(elr_t, fa, fb, src, dst, nheads, hsel_a, hsel_b):
    s_part, p_hbm = _sc_logits(elr_t, src, dst, nheads)
    ra, rb = _sc_messages(fa, fb, p_hbm, src, dst, hsel_a, hsel_b)
    return s_part, ra, rb


def kernel(feats, edge_index, W1, attn_l1, attn_r1, W2, attn_l2, attn_r2):
    src = edge_index[0].astype(jnp.int32)
    dst = edge_index[1].astype(jnp.int32)
    feats_p = jnp.pad(feats, ((0, NP - N), (0, 0)))

    al1 = attn_l1.reshape(H, DH)
    ar1 = attn_r1.reshape(H, DH)
    eye8 = jnp.eye(H, dtype=_F32)
    AL8 = (al1[:, :, None] * eye8[:, None, :]).reshape(D, H)
    AR8 = (ar1[:, :, None] * eye8[:, None, :]).reshape(D, H)
    ALR1 = jnp.concatenate([AL8, AR8], axis=1)
    ALR2 = jnp.zeros((D, 16), _F32)
    ALR2 = ALR2.at[:, 0].set(attn_l2.reshape(D))
    ALR2 = ALR2.at[:, 8].set(attn_r2.reshape(D))

    brep1 = np.zeros((16, D), np.float32)
    for h in range(H):
        brep1[h, h * DH:(h + 1) * DH] = 1.0
    brep1 = jnp.asarray(brep1)
    brep2 = np.zeros((16, D), np.float32)
    brep2[0, :] = 1.0
    brep2 = jnp.asarray(brep2)

    f1a, f1b, elr1 = _tc1(feats_p, W1, ALR1)

    hsel_a1 = [0, 0, 1, 1, 2, 2, 3, 3]
    hsel_b1 = [4, 4, 5, 5, 6, 6, 7, 7]
    s1p, r1a, r1b = _sc_layer(elr1, f1a, f1b, src, dst, H, hsel_a1, hsel_b1)

    h1p, f2a, f2b, elr2 = _tc2(s1p, r1a, r1b, W2, ALR2, brep1)

    hsel2 = [0] * 8
    s2p, r2a, r2b = _sc_layer(elr2, f2a, f2b, src, dst, 1, hsel2, hsel2)

    outp = _tc3(r2a, r2b, s2p, brep2)

    return (feats, h1p[:N], outp[:N])
